# Initial kernel scaffold; baseline (speedup 1.0000x reference)
import jax
import jax.numpy as jnp
from jax.experimental import pallas as pl


def kernel(edge_index, r, z, node_embedding, edge_table, filt_W, filt_b, src_W, out_W1, out_b1, out_W2, out_b2, fc_W1, fc_b1, fc_W2, fc_b2, fc2_W1, fc2_b1, fc2_W2, fc2_b2):
    raise NotImplementedError("write your pallas kernel here")


# speedup vs baseline: 4.8081x; 4.8081x over previous
import functools
import math

import jax
import jax.numpy as jnp
from jax import lax
from jax.experimental import pallas as pl
from jax.experimental.pallas import tpu as pltpu
from jax.experimental.pallas import tpu_sc as plsc

N = 10000
E = 320000
H = 128
R = 64

N_PAD = 10240

NC = 2
NS = 16
NW = NC * NS
EPW = E // NW
KE = 80
NCH = EPW // KE

BE = 2000
NBE = E // BE
BN = 2048
NBN = N_PAD // BN

_LOG2 = math.log(2.0)
_SUB_ROWS = N_PAD // NS


_LOG2E = 1.4426950408889634


def _ssp(x):
    t = jnp.log2(1.0 + jnp.exp2(-jnp.abs(x) * _LOG2E))
    return jnp.maximum(x, 0.0) + (t - 1.0) * _LOG2


def _wid():
    return lax.axis_index("s") * NC + lax.axis_index("c")


@functools.cache
def _sc_kernels():
    mesh = plsc.VectorSubcoreMesh(
        core_axis_name="c", subcore_axis_name="s", num_cores=NC, num_subcores=NS
    )

    @functools.partial(
        pl.kernel,
        out_type=(
            jax.ShapeDtypeStruct((NW, NCH, KE), jnp.int32),
            jax.ShapeDtypeStruct((NW, NCH, KE), jnp.int32),
        ),
        mesh=mesh,
        scratch_types=[
            pltpu.VMEM((NCH, KE), jnp.int32),
            pltpu.VMEM((NCH, KE), jnp.int32),
            pltpu.SemaphoreType.DMA,
        ],
    )
    def _sc_gather_z(src3, dst3, z_hbm, zs3, zd3, idx_v, val_v, sem):
        w = _wid()

        def one(side_idx, side_out):
            pltpu.sync_copy(side_idx.at[w], idx_v)

            def body(j, _):
                pltpu.async_copy(z_hbm.at[idx_v.at[j]], val_v.at[j], sem).wait()
                return 0

            lax.fori_loop(0, NCH, body, 0)
            pltpu.sync_copy(val_v, side_out.at[w])

        one(src3, zs3)
        one(dst3, zd3)

    @functools.partial(
        pl.kernel,
        out_type=jax.ShapeDtypeStruct((E, H), jnp.float32),
        mesh=mesh,
        scratch_types=[
            pltpu.VMEM((NCH, KE), jnp.int32),
            pltpu.VMEM((KE, H), jnp.float32),
            pltpu.VMEM((KE, H), jnp.float32),
            pltpu.SemaphoreType.DMA,
            pltpu.SemaphoreType.DMA,
        ],
    )
    def _sc_gather_rows(hs_hbm, src3, g_hbm, idx_v, buf0, buf1, sem0, sem1):
        w = _wid()
        base = w * EPW
        pltpu.sync_copy(src3.at[w], idx_v)
        pltpu.make_async_copy(hs_hbm.at[idx_v.at[0]], buf0, sem0).start()

        def phase(j, buf, sem, obuf, osem):
            pltpu.make_async_copy(hs_hbm.at[idx_v.at[j]], buf, sem).wait()

            @pl.when(j + 1 < NCH)
            def _():
                pltpu.make_async_copy(hs_hbm.at[idx_v.at[j + 1]], obuf, osem).start()

            pltpu.sync_copy(buf, g_hbm.at[pl.ds(base + j * KE, KE)])

        def body(j, _):
            @pl.when(j % 2 == 0)
            def _():
                phase(j, buf0, sem0, buf1, sem1)

            @pl.when(j % 2 == 1)
            def _():
                phase(j, buf1, sem1, buf0, sem0)

            return 0

        lax.fori_loop(0, NCH, body, 0)

    @functools.partial(
        pl.kernel,
        out_type=jax.ShapeDtypeStruct((NC, N_PAD, H), jnp.float32),
        mesh=mesh,
        scratch_types=[
            pltpu.VMEM((NCH, KE), jnp.int32),
            pltpu.VMEM((KE, H), jnp.float32),
            pltpu.VMEM_SHARED((N_PAD, H), jnp.float32),
            pltpu.SemaphoreType.DMA,
        ],
    )
    def _sc_scatter_add(m_hbm, dst3, zero_hbm, out_hbm, idx_v, buf, acc, sem):
        c = lax.axis_index("c")
        s = lax.axis_index("s")
        w = s * NC + c
        pltpu.sync_copy(
            zero_hbm.at[pl.ds(s * _SUB_ROWS, _SUB_ROWS)],
            acc.at[pl.ds(s * _SUB_ROWS, _SUB_ROWS)],
        )
        pltpu.sync_copy(dst3.at[w], idx_v)
        plsc.subcore_barrier()
        base = w * EPW

        def body(j, _):
            pltpu.sync_copy(m_hbm.at[pl.ds(base + j * KE, KE)], buf)
            pltpu.sync_copy(buf, acc.at[idx_v.at[j]], add=True)
            return 0

        lax.fori_loop(0, NCH, body, 0)
        plsc.subcore_barrier()
        pltpu.sync_copy(
            acc.at[pl.ds(s * _SUB_ROWS, _SUB_ROWS)],
            out_hbm.at[c, pl.ds(s * _SUB_ROWS, _SUB_ROWS)],
        )

    return _sc_gather_z, _sc_gather_rows, _sc_scatter_add


def _cut_body(r_ref, cut_ref):
    r = r_ref[...]
    cut_ref[...] = jnp.where(r < 1.0, 0.5 * (jnp.cos(math.pi * r) + 1.0), 0.0)


E_PADROWS = (E + 511) // 512 * 4


def _cut_call(r2d):
    return pl.pallas_call(
        _cut_body,
        out_shape=jax.ShapeDtypeStruct((E_PADROWS, 128), jnp.float32),
    )(r2d)


def _edge_body(r_ref, zs_ref, zd_ref, cut_ref, g_ref, fW_ref, fb_ref, et_ref, m_ref, *, layer0):
    r = r_ref[0, 0, :].reshape(BE, 1)
    zs = zs_ref[0, 0, :].reshape(BE, 1)
    zd = zd_ref[0, 0, :].reshape(BE, 1)
    delta = 1.0 / (R - 1)
    centers = lax.broadcasted_iota(jnp.int32, (BE, R), 1).astype(jnp.float32) * delta
    t = (r - centers) * (1.0 / delta)
    cutc = cut_ref[0, 0, :].reshape(BE, 1)
    bf = jnp.exp2(t * t * (-0.5 * _LOG2E)) * cutc
    filt = _ssp(
        jnp.dot(bf, fW_ref[...], preferred_element_type=jnp.float32) + fb_ref[...]
    )
    eh = jnp.where(zs == zd, et_ref[1, :][None, :], et_ref[0, :][None, :])
    if layer0:
        g = jnp.sum(g_ref[...], axis=0, keepdims=True)
    else:
        g = g_ref[...]
    m_ref[...] = (g + eh) * filt * cutc


def _edge_call(r3, zs3, zd3, cut3, g_or_w, fW, fb, et, *, layer0):
    vec_spec = pl.BlockSpec((1, 1, BE), lambda i: (i, 0, 0))
    g_spec = (
        pl.BlockSpec((H, H), lambda i: (0, 0))
        if layer0
        else pl.BlockSpec((BE, H), lambda i: (i, 0))
    )
    return pl.pallas_call(
        functools.partial(_edge_body, layer0=layer0),
        grid=(NBE,),
        in_specs=[
            vec_spec,
            vec_spec,
            vec_spec,
            vec_spec,
            g_spec,
            pl.BlockSpec((R, H), lambda i: (0, 0)),
            pl.BlockSpec((1, H), lambda i: (0, 0)),
            pl.BlockSpec((2, H), lambda i: (0, 0)),
        ],
        out_specs=pl.BlockSpec((BE, H), lambda i: (i, 0)),
        out_shape=jax.ShapeDtypeStruct((E, H), jnp.float32),
    )(r3, zs3, zd3, cut3, g_or_w, fW, fb, et)


def _node_body(p_ref, h_ref, W1_ref, b1_ref, W2_ref, b2_ref, sW_ref, hnew_ref, hs_ref, *, layer0):
    agg = p_ref[0] + p_ref[1]
    u = _ssp(jnp.dot(agg, W1_ref[...], preferred_element_type=jnp.float32) + b1_ref[...])
    upd = jnp.dot(u, W2_ref[...], preferred_element_type=jnp.float32) + b2_ref[...]
    if layer0:
        hnew = 1.0 + upd
    else:
        hnew = h_ref[...] + upd
    hnew_ref[...] = hnew
    hs_ref[...] = jnp.dot(hnew, sW_ref[...], preferred_element_type=jnp.float32)


def _node_call(p, h, W1, b1, W2, b2, sW_next, *, layer0):
    full = lambda shape: pl.BlockSpec(shape, lambda i: tuple(0 for _ in shape))
    in_specs = [
        pl.BlockSpec((NC, BN, H), lambda i: (0, i, 0)),
        pl.BlockSpec((BN, H), lambda i: (i, 0)),
        full((H, H)),
        full((1, H)),
        full((H, H)),
        full((1, H)),
        full((H, H)),
    ]
    args = [p, h, W1, b1, W2, b2, sW_next]
    return pl.pallas_call(
        functools.partial(_node_body, layer0=layer0),
        grid=(NBN,),
        in_specs=in_specs,
        out_specs=(
            pl.BlockSpec((BN, H), lambda i: (i, 0)),
            pl.BlockSpec((BN, H), lambda i: (i, 0)),
        ),
        out_shape=(
            jax.ShapeDtypeStruct((N_PAD, H), jnp.float32),
            jax.ShapeDtypeStruct((N_PAD, H), jnp.float32),
        ),
    )(*args)


def _final_body(p_ref, h_ref, W1_ref, b1_ref, W2_ref, b2_ref,
                fW1_ref, fb1_ref, fW2_ref, fb2_ref,
                gW1_ref, gb1_ref, gW2r_ref, gb2_ref, out_ref, acc_ref):
    i = pl.program_id(0)
    agg = p_ref[0] + p_ref[1]
    u = _ssp(jnp.dot(agg, W1_ref[...], preferred_element_type=jnp.float32) + b1_ref[...])
    upd = jnp.dot(u, W2_ref[...], preferred_element_type=jnp.float32) + b2_ref[...]
    hnew = h_ref[...] + upd
    t = _ssp(jnp.dot(hnew, fW1_ref[...], preferred_element_type=jnp.float32) + fb1_ref[...])
    row = lax.broadcasted_iota(jnp.int32, (BN, 1), 0) + i * BN
    t = jnp.where(row < N, t, 0.0)
    part = jnp.sum(t, axis=0, keepdims=True)

    @pl.when(i == 0)
    def _():
        acc_ref[...] = part

    @pl.when(i > 0)
    def _():
        acc_ref[...] = acc_ref[...] + part

    @pl.when(i == NBN - 1)
    def _():
        pooled = (
            jnp.dot(acc_ref[...] * (1.0 / N), fW2_ref[...],
                    preferred_element_type=jnp.float32)
            + fb2_ref[...]
        )
        o = _ssp(
            jnp.dot(pooled, gW1_ref[...], preferred_element_type=jnp.float32)
            + gb1_ref[...]
        )
        out_ref[...] = (
            jnp.sum(o * gW2r_ref[...], axis=1, keepdims=True) + gb2_ref[...]
        )


def _final_call(p, h, W1, b1, W2, b2, fW1, fb1, fW2, fb2, gW1, gb1, gW2r, gb2):
    full = lambda shape: pl.BlockSpec(shape, lambda i: tuple(0 for _ in shape))
    return pl.pallas_call(
        _final_body,
        grid=(NBN,),
        in_specs=[
            pl.BlockSpec((NC, BN, H), lambda i: (0, i, 0)),
            pl.BlockSpec((BN, H), lambda i: (i, 0)),
            full((H, H)), full((1, H)), full((H, H)), full((1, H)),
            full((H, H)), full((1, H)), full((H, H)), full((1, H)),
            full((H, H)), full((1, H)), full((1, H)), full((1, 1)),
        ],
        out_specs=pl.BlockSpec((1, 1), lambda i: (0, 0)),
        out_shape=jax.ShapeDtypeStruct((1, 1), jnp.float32),
        scratch_shapes=[pltpu.VMEM((1, H), jnp.float32)],
    )(p, h, W1, b1, W2, b2, fW1, fb1, fW2, fb2, gW1, gb1, gW2r, gb2)


def kernel(edge_index, r, z, node_embedding, edge_table, filt_W, filt_b, src_W,
           out_W1, out_b1, out_W2, out_b2, fc_W1, fc_b1, fc_W2, fc_b2,
           fc2_W1, fc2_b1, fc2_W2, fc2_b2):
    _sc_gather_z, _sc_gather_rows, _sc_scatter_add = _sc_kernels()
    src3 = edge_index[0].reshape(NW, NCH, KE)
    dst3 = edge_index[1].reshape(NW, NCH, KE)
    r3 = r.reshape(NBE, 1, BE)
    zeros_pad = jnp.zeros((N_PAD, H), jnp.float32)

    rpad = jnp.concatenate([r, jnp.zeros((E_PADROWS * 128 - E,), jnp.float32)])
    cut3 = _cut_call(rpad.reshape(E_PADROWS, 128)).reshape(-1)[:E].reshape(NBE, 1, BE)

    zs3, zd3 = _sc_gather_z(src3, dst3, z)
    zs3e = zs3.reshape(NBE, 1, BE)
    zd3e = zd3.reshape(NBE, 1, BE)

    def edge_table_args(l):
        return (filt_W[l], filt_b[l].reshape(1, H), edge_table)

    m = _edge_call(r3, zs3e, zd3e, cut3, src_W[0], *edge_table_args(0), layer0=True)
    p = _sc_scatter_add(m, dst3, zeros_pad)
    h, hs = _node_call(p, zeros_pad, out_W1[0], out_b1[0].reshape(1, H),
                       out_W2[0], out_b2[0].reshape(1, H), src_W[1], layer0=True)

    g = _sc_gather_rows(hs, src3)
    m = _edge_call(r3, zs3e, zd3e, cut3, g, *edge_table_args(1), layer0=False)
    p = _sc_scatter_add(m, dst3, zeros_pad)
    h, hs = _node_call(p, h, out_W1[1], out_b1[1].reshape(1, H),
                       out_W2[1], out_b2[1].reshape(1, H), src_W[2], layer0=False)

    g = _sc_gather_rows(hs, src3)
    m = _edge_call(r3, zs3e, zd3e, cut3, g, *edge_table_args(2), layer0=False)
    p = _sc_scatter_add(m, dst3, zeros_pad)

    out = _final_call(
        p, h, out_W1[2], out_b1[2].reshape(1, H), out_W2[2], out_b2[2].reshape(1, H),
        fc_W1, fc_b1.reshape(1, H), fc_W2, fc_b2.reshape(1, H),
        fc2_W1, fc2_b1.reshape(1, H), fc2_W2.reshape(1, H), fc2_b2.reshape(1, 1),
    )
    return out.reshape(1)


# speedup vs baseline: 6.2808x; 1.3063x over previous
import functools
import math

import jax
import jax.numpy as jnp
from jax import lax
from jax.experimental import pallas as pl
from jax.experimental.pallas import tpu as pltpu
from jax.experimental.pallas import tpu_sc as plsc

N = 10000
E = 320000
H = 128
R = 64

N_PAD = 10240

NC = 2
NS = 16
NW = NC * NS
EPW = E // NW
KE = 80
NCH = EPW // KE

BE = 2000
NBE = E // BE
BN = 2048
NBN = N_PAD // BN

_LOG2 = math.log(2.0)
_SUB_ROWS = N_PAD // NS


_LOG2E = 1.4426950408889634


def _ssp(x):
    t = jnp.log2(1.0 + jnp.exp2(-jnp.abs(x) * _LOG2E))
    return jnp.maximum(x, 0.0) + (t - 1.0) * _LOG2


def _wid():
    return lax.axis_index("s") * NC + lax.axis_index("c")


@functools.cache
def _sc_kernels():
    mesh = plsc.VectorSubcoreMesh(
        core_axis_name="c", subcore_axis_name="s", num_cores=NC, num_subcores=NS
    )

    @functools.partial(
        pl.kernel,
        out_type=(
            jax.ShapeDtypeStruct((NW, NCH, KE), jnp.int32),
            jax.ShapeDtypeStruct((NW, NCH, KE), jnp.int32),
        ),
        mesh=mesh,
        scratch_types=[
            pltpu.VMEM((NCH, KE), jnp.int32),
            pltpu.VMEM((NCH, KE), jnp.int32),
            pltpu.VMEM((NCH, KE), jnp.int32),
            pltpu.VMEM((NCH, KE), jnp.int32),
            pltpu.SemaphoreType.DMA,
            pltpu.SemaphoreType.DMA,
        ],
    )
    def _sc_gather_z(src3, dst3, z_hbm, zs3, zd3, sidx, didx, sval, dval, ssem, dsem):
        w = _wid()
        pltpu.sync_copy(src3.at[w], sidx)
        pltpu.sync_copy(dst3.at[w], didx)
        PIPE = 8

        def fire(j, idx, val, sem):
            pltpu.make_async_copy(z_hbm.at[idx.at[j]], val.at[j], sem).start()

        def drain(j, idx, val, sem):
            pltpu.make_async_copy(z_hbm.at[idx.at[j]], val.at[j], sem).wait()

        def body(j, _):
            fire(j, sidx, sval, ssem)
            fire(j, didx, dval, dsem)

            @pl.when(j >= PIPE)
            def _():
                drain(j - PIPE, sidx, sval, ssem)
                drain(j - PIPE, didx, dval, dsem)

            return 0

        lax.fori_loop(0, NCH, body, 0)

        def tail(j, _):
            drain(j, sidx, sval, ssem)
            drain(j, didx, dval, dsem)
            return 0

        lax.fori_loop(NCH - PIPE, NCH, tail, 0)
        pltpu.sync_copy(sval, zs3.at[w])
        pltpu.sync_copy(dval, zd3.at[w])

    NB_G = 5
    NO_G = NCH // NB_G

    @functools.partial(
        pl.kernel,
        out_type=jax.ShapeDtypeStruct((E, H), jnp.float32),
        mesh=mesh,
        scratch_types=[
            pltpu.VMEM((NCH, KE), jnp.int32),
            pltpu.VMEM((KE, H), jnp.float32),
            pltpu.VMEM((KE, H), jnp.float32),
            pltpu.VMEM((KE, H), jnp.float32),
            pltpu.VMEM((KE, H), jnp.float32),
            pltpu.VMEM((KE, H), jnp.float32),
            pltpu.SemaphoreType.DMA,
            pltpu.SemaphoreType.DMA,
            pltpu.SemaphoreType.DMA,
            pltpu.SemaphoreType.DMA,
            pltpu.SemaphoreType.DMA,
            pltpu.SemaphoreType.DMA,
            pltpu.SemaphoreType.DMA,
            pltpu.SemaphoreType.DMA,
            pltpu.SemaphoreType.DMA,
            pltpu.SemaphoreType.DMA,
        ],
    )
    def _sc_gather_rows(hs_hbm, src3, g_hbm, idx_v, b0, b1, b2, b3, b4,
                        g0, g1, g2, g3, g4, w0, w1, w2, w3, w4):
        bufs = (b0, b1, b2, b3, b4)
        gsems = (g0, g1, g2, g3, g4)
        wsems = (w0, w1, w2, w3, w4)
        w = _wid()
        base = w * EPW
        pltpu.sync_copy(src3.at[w], idx_v)

        def outer(o, _):
            @pl.when(o > 0)
            def _():
                for b in range(NB_G):
                    pltpu.make_async_copy(
                        bufs[b],
                        g_hbm.at[pl.ds(base + ((o - 1) * NB_G + b) * KE, KE)],
                        wsems[b],
                    ).wait()

            for b in range(NB_G):
                pltpu.make_async_copy(
                    hs_hbm.at[idx_v.at[o * NB_G + b]], bufs[b], gsems[b]
                ).start()
            for b in range(NB_G):
                pltpu.make_async_copy(
                    hs_hbm.at[idx_v.at[o * NB_G + b]], bufs[b], gsems[b]
                ).wait()
                pltpu.make_async_copy(
                    bufs[b],
                    g_hbm.at[pl.ds(base + (o * NB_G + b) * KE, KE)],
                    wsems[b],
                ).start()

            return 0

        lax.fori_loop(0, NO_G, outer, 0)
        for b in range(NB_G):
            pltpu.make_async_copy(
                bufs[b],
                g_hbm.at[pl.ds(base + ((NO_G - 1) * NB_G + b) * KE, KE)],
                wsems[b],
            ).wait()

    @functools.partial(
        pl.kernel,
        out_type=jax.ShapeDtypeStruct((NC, N_PAD, H), jnp.float32),
        mesh=mesh,
        scratch_types=[
            pltpu.VMEM((NCH, KE), jnp.int32),
            pltpu.VMEM((KE, H), jnp.float32),
            pltpu.VMEM((KE, H), jnp.float32),
            pltpu.VMEM_SHARED((N_PAD, H), jnp.float32),
            pltpu.SemaphoreType.DMA,
            pltpu.SemaphoreType.DMA,
        ],
    )
    def _sc_scatter_add(m_hbm, dst3, zero_hbm, out_hbm, idx_v, b0, b1,
                        acc, l0, l1):
        bufs = (b0, b1)
        lsems = (l0, l1)
        c = lax.axis_index("c")
        s = lax.axis_index("s")
        w = s * NC + c
        pltpu.sync_copy(
            zero_hbm.at[pl.ds(s * _SUB_ROWS, _SUB_ROWS)],
            acc.at[pl.ds(s * _SUB_ROWS, _SUB_ROWS)],
        )
        pltpu.sync_copy(dst3.at[w], idx_v)
        plsc.subcore_barrier()
        base = w * EPW

        def load(j, slot):
            pltpu.make_async_copy(
                m_hbm.at[pl.ds(base + j * KE, KE)], bufs[slot], lsems[slot]
            ).start()

        def wait_load(j, slot):
            pltpu.make_async_copy(
                m_hbm.at[pl.ds(base + j * KE, KE)], bufs[slot], lsems[slot]
            ).wait()

        load(0, 0)

        def body(j, _):
            @pl.when(j % 2 == 0)
            def _():
                @pl.when(j + 1 < NCH)
                def _():
                    load(j + 1, 1)

                wait_load(j, 0)
                pltpu.sync_copy(bufs[0], acc.at[idx_v.at[j]], add=True)

            @pl.when(j % 2 == 1)
            def _():
                @pl.when(j + 1 < NCH)
                def _():
                    load(j + 1, 0)

                wait_load(j, 1)
                pltpu.sync_copy(bufs[1], acc.at[idx_v.at[j]], add=True)

            return 0

        lax.fori_loop(0, NCH, body, 0)
        plsc.subcore_barrier()
        pltpu.sync_copy(
            acc.at[pl.ds(s * _SUB_ROWS, _SUB_ROWS)],
            out_hbm.at[c, pl.ds(s * _SUB_ROWS, _SUB_ROWS)],
        )

    return _sc_gather_z, _sc_gather_rows, _sc_scatter_add


def _cut_body(r_ref, cut_ref):
    r = r_ref[...]
    cut_ref[...] = jnp.where(r < 1.0, 0.5 * (jnp.cos(math.pi * r) + 1.0), 0.0)


E_PADROWS = (E + 511) // 512 * 4


def _cut_call(r2d):
    return pl.pallas_call(
        _cut_body,
        out_shape=jax.ShapeDtypeStruct((E_PADROWS, 128), jnp.float32),
    )(r2d)


def _edge_body(r_ref, zs_ref, zd_ref, cut_ref, g_ref, fW_ref, fb_ref, et_ref, m_ref, *, layer0):
    r = r_ref[0, 0, :].reshape(BE, 1)
    zs = zs_ref[0, 0, :].reshape(BE, 1)
    zd = zd_ref[0, 0, :].reshape(BE, 1)
    delta = 1.0 / (R - 1)
    centers = lax.broadcasted_iota(jnp.int32, (BE, R), 1).astype(jnp.float32) * delta
    t = (r - centers) * (1.0 / delta)
    cutc = cut_ref[0, 0, :].reshape(BE, 1)
    bf = jnp.exp2(t * t * (-0.5 * _LOG2E)) * cutc
    filt = _ssp(
        jnp.dot(bf, fW_ref[...], preferred_element_type=jnp.float32) + fb_ref[...]
    )
    eh = jnp.where(zs == zd, et_ref[1, :][None, :], et_ref[0, :][None, :])
    if layer0:
        g = jnp.sum(g_ref[...], axis=0, keepdims=True)
    else:
        g = g_ref[...]
    m_ref[...] = (g + eh) * filt * cutc


def _edge_call(r3, zs3, zd3, cut3, g_or_w, fW, fb, et, *, layer0):
    vec_spec = pl.BlockSpec((1, 1, BE), lambda i: (i, 0, 0))
    g_spec = (
        pl.BlockSpec((H, H), lambda i: (0, 0))
        if layer0
        else pl.BlockSpec((BE, H), lambda i: (i, 0))
    )
    return pl.pallas_call(
        functools.partial(_edge_body, layer0=layer0),
        grid=(NBE,),
        in_specs=[
            vec_spec,
            vec_spec,
            vec_spec,
            vec_spec,
            g_spec,
            pl.BlockSpec((R, H), lambda i: (0, 0)),
            pl.BlockSpec((1, H), lambda i: (0, 0)),
            pl.BlockSpec((2, H), lambda i: (0, 0)),
        ],
        out_specs=pl.BlockSpec((BE, H), lambda i: (i, 0)),
        out_shape=jax.ShapeDtypeStruct((E, H), jnp.float32),
    )(r3, zs3, zd3, cut3, g_or_w, fW, fb, et)


def _node_body(p_ref, h_ref, W1_ref, b1_ref, W2_ref, b2_ref, sW_ref, hnew_ref, hs_ref, *, layer0):
    agg = p_ref[0] + p_ref[1]
    u = _ssp(jnp.dot(agg, W1_ref[...], preferred_element_type=jnp.float32) + b1_ref[...])
    upd = jnp.dot(u, W2_ref[...], preferred_element_type=jnp.float32) + b2_ref[...]
    if layer0:
        hnew = 1.0 + upd
    else:
        hnew = h_ref[...] + upd
    hnew_ref[...] = hnew
    hs_ref[...] = jnp.dot(hnew, sW_ref[...], preferred_element_type=jnp.float32)


def _node_call(p, h, W1, b1, W2, b2, sW_next, *, layer0):
    full = lambda shape: pl.BlockSpec(shape, lambda i: tuple(0 for _ in shape))
    in_specs = [
        pl.BlockSpec((NC, BN, H), lambda i: (0, i, 0)),
        pl.BlockSpec((BN, H), lambda i: (i, 0)),
        full((H, H)),
        full((1, H)),
        full((H, H)),
        full((1, H)),
        full((H, H)),
    ]
    args = [p, h, W1, b1, W2, b2, sW_next]
    return pl.pallas_call(
        functools.partial(_node_body, layer0=layer0),
        grid=(NBN,),
        in_specs=in_specs,
        out_specs=(
            pl.BlockSpec((BN, H), lambda i: (i, 0)),
            pl.BlockSpec((BN, H), lambda i: (i, 0)),
        ),
        out_shape=(
            jax.ShapeDtypeStruct((N_PAD, H), jnp.float32),
            jax.ShapeDtypeStruct((N_PAD, H), jnp.float32),
        ),
    )(*args)


def _final_body(p_ref, h_ref, W1_ref, b1_ref, W2_ref, b2_ref,
                fW1_ref, fb1_ref, fW2_ref, fb2_ref,
                gW1_ref, gb1_ref, gW2r_ref, gb2_ref, out_ref, acc_ref):
    i = pl.program_id(0)
    agg = p_ref[0] + p_ref[1]
    u = _ssp(jnp.dot(agg, W1_ref[...], preferred_element_type=jnp.float32) + b1_ref[...])
    upd = jnp.dot(u, W2_ref[...], preferred_element_type=jnp.float32) + b2_ref[...]
    hnew = h_ref[...] + upd
    t = _ssp(jnp.dot(hnew, fW1_ref[...], preferred_element_type=jnp.float32) + fb1_ref[...])
    row = lax.broadcasted_iota(jnp.int32, (BN, 1), 0) + i * BN
    t = jnp.where(row < N, t, 0.0)
    part = jnp.sum(t, axis=0, keepdims=True)

    @pl.when(i == 0)
    def _():
        acc_ref[...] = part

    @pl.when(i > 0)
    def _():
        acc_ref[...] = acc_ref[...] + part

    @pl.when(i == NBN - 1)
    def _():
        pooled = (
            jnp.dot(acc_ref[...] * (1.0 / N), fW2_ref[...],
                    preferred_element_type=jnp.float32)
            + fb2_ref[...]
        )
        o = _ssp(
            jnp.dot(pooled, gW1_ref[...], preferred_element_type=jnp.float32)
            + gb1_ref[...]
        )
        out_ref[...] = (
            jnp.sum(o * gW2r_ref[...], axis=1, keepdims=True) + gb2_ref[...]
        )


def _final_call(p, h, W1, b1, W2, b2, fW1, fb1, fW2, fb2, gW1, gb1, gW2r, gb2):
    full = lambda shape: pl.BlockSpec(shape, lambda i: tuple(0 for _ in shape))
    return pl.pallas_call(
        _final_body,
        grid=(NBN,),
        in_specs=[
            pl.BlockSpec((NC, BN, H), lambda i: (0, i, 0)),
            pl.BlockSpec((BN, H), lambda i: (i, 0)),
            full((H, H)), full((1, H)), full((H, H)), full((1, H)),
            full((H, H)), full((1, H)), full((H, H)), full((1, H)),
            full((H, H)), full((1, H)), full((1, H)), full((1, 1)),
        ],
        out_specs=pl.BlockSpec((1, 1), lambda i: (0, 0)),
        out_shape=jax.ShapeDtypeStruct((1, 1), jnp.float32),
        scratch_shapes=[pltpu.VMEM((1, H), jnp.float32)],
    )(p, h, W1, b1, W2, b2, fW1, fb1, fW2, fb2, gW1, gb1, gW2r, gb2)


def kernel(edge_index, r, z, node_embedding, edge_table, filt_W, filt_b, src_W,
           out_W1, out_b1, out_W2, out_b2, fc_W1, fc_b1, fc_W2, fc_b2,
           fc2_W1, fc2_b1, fc2_W2, fc2_b2):
    _sc_gather_z, _sc_gather_rows, _sc_scatter_add = _sc_kernels()
    src3 = edge_index[0].reshape(NW, NCH, KE)
    dst3 = edge_index[1].reshape(NW, NCH, KE)
    r3 = r.reshape(NBE, 1, BE)
    zeros_pad = jnp.zeros((N_PAD, H), jnp.float32)

    rpad = jnp.concatenate([r, jnp.zeros((E_PADROWS * 128 - E,), jnp.float32)])
    cut3 = _cut_call(rpad.reshape(E_PADROWS, 128)).reshape(-1)[:E].reshape(NBE, 1, BE)

    zs3, zd3 = _sc_gather_z(src3, dst3, z)
    zs3e = zs3.reshape(NBE, 1, BE)
    zd3e = zd3.reshape(NBE, 1, BE)

    def edge_table_args(l):
        return (filt_W[l], filt_b[l].reshape(1, H), edge_table)

    m = _edge_call(r3, zs3e, zd3e, cut3, src_W[0], *edge_table_args(0), layer0=True)
    p = _sc_scatter_add(m, dst3, zeros_pad)
    h, hs = _node_call(p, zeros_pad, out_W1[0], out_b1[0].reshape(1, H),
                       out_W2[0], out_b2[0].reshape(1, H), src_W[1], layer0=True)

    g = _sc_gather_rows(hs, src3)
    m = _edge_call(r3, zs3e, zd3e, cut3, g, *edge_table_args(1), layer0=False)
    p = _sc_scatter_add(m, dst3, zeros_pad)
    h, hs = _node_call(p, h, out_W1[1], out_b1[1].reshape(1, H),
                       out_W2[1], out_b2[1].reshape(1, H), src_W[2], layer0=False)

    g = _sc_gather_rows(hs, src3)
    m = _edge_call(r3, zs3e, zd3e, cut3, g, *edge_table_args(2), layer0=False)
    p = _sc_scatter_add(m, dst3, zeros_pad)

    out = _final_call(
        p, h, out_W1[2], out_b1[2].reshape(1, H), out_W2[2], out_b2[2].reshape(1, H),
        fc_W1, fc_b1.reshape(1, H), fc_W2, fc_b2.reshape(1, H),
        fc2_W1, fc2_b1.reshape(1, H), fc2_W2.reshape(1, H), fc2_b2.reshape(1, 1),
    )
    return out.reshape(1)


# speedup vs baseline: 7.1529x; 1.1388x over previous
import functools
import math

import jax
import jax.numpy as jnp
from jax import lax
from jax.experimental import pallas as pl
from jax.experimental.pallas import tpu as pltpu
from jax.experimental.pallas import tpu_sc as plsc

N = 10000
E = 320000
H = 128
R = 64

N_PAD = 10240

NC = 2
NS = 16
NW = NC * NS
EPW = E // NW
KE = 80
NCH = EPW // KE

BE = 2000
NBE = E // BE
BN = 2048
NBN = N_PAD // BN

_LOG2 = math.log(2.0)
_SUB_ROWS = N_PAD // NS


_LOG2E = 1.4426950408889634


def _ssp(x):
    t = jnp.log2(1.0 + jnp.exp2(-jnp.abs(x) * _LOG2E))
    return jnp.maximum(x, 0.0) + (t - 1.0) * _LOG2


def _wid():
    return lax.axis_index("s") * NC + lax.axis_index("c")


@functools.cache
def _sc_kernels():
    mesh = plsc.VectorSubcoreMesh(
        core_axis_name="c", subcore_axis_name="s", num_cores=NC, num_subcores=NS
    )

    @functools.partial(
        pl.kernel,
        out_type=(
            jax.ShapeDtypeStruct((NW, NCH, KE), jnp.int32),
            jax.ShapeDtypeStruct((NW, NCH, KE), jnp.int32),
        ),
        mesh=mesh,
        scratch_types=[
            pltpu.VMEM((NCH, KE), jnp.int32),
            pltpu.VMEM((NCH, KE), jnp.int32),
            pltpu.VMEM((NCH, KE), jnp.int32),
            pltpu.VMEM((NCH, KE), jnp.int32),
            pltpu.SemaphoreType.DMA,
            pltpu.SemaphoreType.DMA,
        ],
    )
    def _sc_gather_z(src3, dst3, z_hbm, zs3, zd3, sidx, didx, sval, dval, ssem, dsem):
        w = _wid()
        pltpu.sync_copy(src3.at[w], sidx)
        pltpu.sync_copy(dst3.at[w], didx)
        PIPE = 8

        def fire(j, idx, val, sem):
            pltpu.make_async_copy(z_hbm.at[idx.at[j]], val.at[j], sem).start()

        def drain(j, idx, val, sem):
            pltpu.make_async_copy(z_hbm.at[idx.at[j]], val.at[j], sem).wait()

        def body(j, _):
            fire(j, sidx, sval, ssem)
            fire(j, didx, dval, dsem)

            @pl.when(j >= PIPE)
            def _():
                drain(j - PIPE, sidx, sval, ssem)
                drain(j - PIPE, didx, dval, dsem)

            return 0

        lax.fori_loop(0, NCH, body, 0)

        def tail(j, _):
            drain(j, sidx, sval, ssem)
            drain(j, didx, dval, dsem)
            return 0

        lax.fori_loop(NCH - PIPE, NCH, tail, 0)
        pltpu.sync_copy(sval, zs3.at[w])
        pltpu.sync_copy(dval, zd3.at[w])

    NB_G = 5
    NO_G = NCH // NB_G

    @functools.partial(
        pl.kernel,
        out_type=jax.ShapeDtypeStruct((E, H), jnp.float32),
        mesh=mesh,
        scratch_types=[
            pltpu.VMEM((NCH, KE), jnp.int32),
            pltpu.VMEM((KE, H), jnp.float32),
            pltpu.VMEM((KE, H), jnp.float32),
            pltpu.VMEM((KE, H), jnp.float32),
            pltpu.VMEM((KE, H), jnp.float32),
            pltpu.VMEM((KE, H), jnp.float32),
            pltpu.SemaphoreType.DMA,
            pltpu.SemaphoreType.DMA,
            pltpu.SemaphoreType.DMA,
            pltpu.SemaphoreType.DMA,
            pltpu.SemaphoreType.DMA,
            pltpu.SemaphoreType.DMA,
            pltpu.SemaphoreType.DMA,
            pltpu.SemaphoreType.DMA,
            pltpu.SemaphoreType.DMA,
            pltpu.SemaphoreType.DMA,
        ],
    )
    def _sc_gather_rows(hs_hbm, src3, g_hbm, idx_v, b0, b1, b2, b3, b4,
                        g0, g1, g2, g3, g4, w0, w1, w2, w3, w4):
        bufs = (b0, b1, b2, b3, b4)
        gsems = (g0, g1, g2, g3, g4)
        wsems = (w0, w1, w2, w3, w4)
        w = _wid()
        base = w * EPW
        pltpu.sync_copy(src3.at[w], idx_v)

        def outer(o, _):
            @pl.when(o > 0)
            def _():
                for b in range(NB_G):
                    pltpu.make_async_copy(
                        bufs[b],
                        g_hbm.at[pl.ds(base + ((o - 1) * NB_G + b) * KE, KE)],
                        wsems[b],
                    ).wait()

            for b in range(NB_G):
                pltpu.make_async_copy(
                    hs_hbm.at[idx_v.at[o * NB_G + b]], bufs[b], gsems[b]
                ).start()
            for b in range(NB_G):
                pltpu.make_async_copy(
                    hs_hbm.at[idx_v.at[o * NB_G + b]], bufs[b], gsems[b]
                ).wait()
                pltpu.make_async_copy(
                    bufs[b],
                    g_hbm.at[pl.ds(base + (o * NB_G + b) * KE, KE)],
                    wsems[b],
                ).start()

            return 0

        lax.fori_loop(0, NO_G, outer, 0)
        for b in range(NB_G):
            pltpu.make_async_copy(
                bufs[b],
                g_hbm.at[pl.ds(base + ((NO_G - 1) * NB_G + b) * KE, KE)],
                wsems[b],
            ).wait()

    @functools.partial(
        pl.kernel,
        out_type=jax.ShapeDtypeStruct((NC, N_PAD, H), jnp.float32),
        mesh=mesh,
        scratch_types=[
            pltpu.VMEM((NCH, KE), jnp.int32),
            pltpu.VMEM((KE, H), jnp.float32),
            pltpu.VMEM((KE, H), jnp.float32),
            pltpu.VMEM_SHARED((N_PAD, H), jnp.float32),
            pltpu.SemaphoreType.DMA,
            pltpu.SemaphoreType.DMA,
        ],
    )
    def _sc_scatter_add(m_hbm, dst3, zero_hbm, out_hbm, idx_v, b0, b1,
                        acc, l0, l1):
        bufs = (b0, b1)
        lsems = (l0, l1)
        c = lax.axis_index("c")
        s = lax.axis_index("s")
        w = s * NC + c
        pltpu.sync_copy(
            zero_hbm.at[pl.ds(s * _SUB_ROWS, _SUB_ROWS)],
            acc.at[pl.ds(s * _SUB_ROWS, _SUB_ROWS)],
        )
        pltpu.sync_copy(dst3.at[w], idx_v)
        plsc.subcore_barrier()
        base = w * EPW

        def load(j, slot):
            pltpu.make_async_copy(
                m_hbm.at[pl.ds(base + j * KE, KE)], bufs[slot], lsems[slot]
            ).start()

        def wait_load(j, slot):
            pltpu.make_async_copy(
                m_hbm.at[pl.ds(base + j * KE, KE)], bufs[slot], lsems[slot]
            ).wait()

        load(0, 0)

        def body(j, _):
            @pl.when(j % 2 == 0)
            def _():
                @pl.when(j + 1 < NCH)
                def _():
                    load(j + 1, 1)

                wait_load(j, 0)
                pltpu.sync_copy(bufs[0], acc.at[idx_v.at[j]], add=True)

            @pl.when(j % 2 == 1)
            def _():
                @pl.when(j + 1 < NCH)
                def _():
                    load(j + 1, 0)

                wait_load(j, 1)
                pltpu.sync_copy(bufs[1], acc.at[idx_v.at[j]], add=True)

            return 0

        lax.fori_loop(0, NCH, body, 0)
        plsc.subcore_barrier()
        pltpu.sync_copy(
            acc.at[pl.ds(s * _SUB_ROWS, _SUB_ROWS)],
            out_hbm.at[c, pl.ds(s * _SUB_ROWS, _SUB_ROWS)],
        )

    return _sc_gather_z, _sc_gather_rows, _sc_scatter_add


def _cut_body(r_ref, zs_ref, zd_ref, pk_ref):
    r = r_ref[...]
    cut = jnp.where(r < 1.0, 0.5 * (jnp.cos(math.pi * r) + 1.0), 0.0)
    pk_ref[...] = jnp.where(zs_ref[...] == zd_ref[...], -cut, cut)


E_PADROWS = (E + 511) // 512 * 4


def _cut_call(r2d, zs2d, zd2d):
    return pl.pallas_call(
        _cut_body,
        out_shape=jax.ShapeDtypeStruct((E_PADROWS, 128), jnp.float32),
    )(r2d, zs2d, zd2d)


def _edge_body(r_ref, pk_ref, g_ref, fW_ref, fb_ref, et_ref, m_ref, *, layer0):
    r = r_ref[0, 0, :].reshape(1, BE)
    pc = pk_ref[0, 0, :].reshape(BE, 1)
    cutc = jnp.abs(pc)
    delta = 1.0 / (R - 1)
    centers = lax.broadcasted_iota(jnp.int32, (R, BE), 0).astype(jnp.float32) * delta
    t = (r - centers) * (1.0 / delta)
    bfT = jnp.exp2(t * t * (-0.5 * _LOG2E)) * jnp.abs(pk_ref[0, 0, :].reshape(1, BE))
    filt = _ssp(
        lax.dot_general(bfT, fW_ref[...], (((0,), (0,)), ((), ())),
                        preferred_element_type=jnp.float32)
        + fb_ref[...]
    )
    eh = jnp.where(pc < 0.0, et_ref[1, :][None, :], et_ref[0, :][None, :])
    if layer0:
        g = jnp.sum(g_ref[...], axis=0, keepdims=True)
    else:
        g = g_ref[...]
    m_ref[...] = (g + eh) * filt * cutc


def _edge_call(r3, pk3, g_or_w, fW, fb, et, *, layer0):
    vec_spec = pl.BlockSpec((1, 1, BE), lambda i: (i, 0, 0))
    g_spec = (
        pl.BlockSpec((H, H), lambda i: (0, 0))
        if layer0
        else pl.BlockSpec((BE, H), lambda i: (i, 0))
    )
    return pl.pallas_call(
        functools.partial(_edge_body, layer0=layer0),
        grid=(NBE,),
        in_specs=[
            vec_spec,
            vec_spec,
            g_spec,
            pl.BlockSpec((R, H), lambda i: (0, 0)),
            pl.BlockSpec((1, H), lambda i: (0, 0)),
            pl.BlockSpec((2, H), lambda i: (0, 0)),
        ],
        out_specs=pl.BlockSpec((BE, H), lambda i: (i, 0)),
        out_shape=jax.ShapeDtypeStruct((E, H), jnp.float32),
    )(r3, pk3, g_or_w, fW, fb, et)


def _node_body(p_ref, h_ref, W1_ref, b1_ref, W2_ref, b2_ref, sW_ref, hnew_ref, hs_ref, *, layer0):
    agg = p_ref[0] + p_ref[1]
    u = _ssp(jnp.dot(agg, W1_ref[...], preferred_element_type=jnp.float32) + b1_ref[...])
    upd = jnp.dot(u, W2_ref[...], preferred_element_type=jnp.float32) + b2_ref[...]
    if layer0:
        hnew = 1.0 + upd
    else:
        hnew = h_ref[...] + upd
    hnew_ref[...] = hnew
    hs_ref[...] = jnp.dot(hnew, sW_ref[...], preferred_element_type=jnp.float32)


def _node_call(p, h, W1, b1, W2, b2, sW_next, *, layer0):
    full = lambda shape: pl.BlockSpec(shape, lambda i: tuple(0 for _ in shape))
    in_specs = [
        pl.BlockSpec((NC, BN, H), lambda i: (0, i, 0)),
        pl.BlockSpec((BN, H), lambda i: (i, 0)),
        full((H, H)),
        full((1, H)),
        full((H, H)),
        full((1, H)),
        full((H, H)),
    ]
    args = [p, h, W1, b1, W2, b2, sW_next]
    return pl.pallas_call(
        functools.partial(_node_body, layer0=layer0),
        grid=(NBN,),
        in_specs=in_specs,
        out_specs=(
            pl.BlockSpec((BN, H), lambda i: (i, 0)),
            pl.BlockSpec((BN, H), lambda i: (i, 0)),
        ),
        out_shape=(
            jax.ShapeDtypeStruct((N_PAD, H), jnp.float32),
            jax.ShapeDtypeStruct((N_PAD, H), jnp.float32),
        ),
    )(*args)


def _final_body(p_ref, h_ref, W1_ref, b1_ref, W2_ref, b2_ref,
                fW1_ref, fb1_ref, fW2_ref, fb2_ref,
                gW1_ref, gb1_ref, gW2r_ref, gb2_ref, out_ref, acc_ref):
    i = pl.program_id(0)
    agg = p_ref[0] + p_ref[1]
    u = _ssp(jnp.dot(agg, W1_ref[...], preferred_element_type=jnp.float32) + b1_ref[...])
    upd = jnp.dot(u, W2_ref[...], preferred_element_type=jnp.float32) + b2_ref[...]
    hnew = h_ref[...] + upd
    t = _ssp(jnp.dot(hnew, fW1_ref[...], preferred_element_type=jnp.float32) + fb1_ref[...])
    row = lax.broadcasted_iota(jnp.int32, (BN, 1), 0) + i * BN
    t = jnp.where(row < N, t, 0.0)
    part = jnp.sum(t, axis=0, keepdims=True)

    @pl.when(i == 0)
    def _():
        acc_ref[...] = part

    @pl.when(i > 0)
    def _():
        acc_ref[...] = acc_ref[...] + part

    @pl.when(i == NBN - 1)
    def _():
        pooled = (
            jnp.dot(acc_ref[...] * (1.0 / N), fW2_ref[...],
                    preferred_element_type=jnp.float32)
            + fb2_ref[...]
        )
        o = _ssp(
            jnp.dot(pooled, gW1_ref[...], preferred_element_type=jnp.float32)
            + gb1_ref[...]
        )
        out_ref[...] = (
            jnp.sum(o * gW2r_ref[...], axis=1, keepdims=True) + gb2_ref[...]
        )


def _final_call(p, h, W1, b1, W2, b2, fW1, fb1, fW2, fb2, gW1, gb1, gW2r, gb2):
    full = lambda shape: pl.BlockSpec(shape, lambda i: tuple(0 for _ in shape))
    return pl.pallas_call(
        _final_body,
        grid=(NBN,),
        in_specs=[
            pl.BlockSpec((NC, BN, H), lambda i: (0, i, 0)),
            pl.BlockSpec((BN, H), lambda i: (i, 0)),
            full((H, H)), full((1, H)), full((H, H)), full((1, H)),
            full((H, H)), full((1, H)), full((H, H)), full((1, H)),
            full((H, H)), full((1, H)), full((1, H)), full((1, 1)),
        ],
        out_specs=pl.BlockSpec((1, 1), lambda i: (0, 0)),
        out_shape=jax.ShapeDtypeStruct((1, 1), jnp.float32),
        scratch_shapes=[pltpu.VMEM((1, H), jnp.float32)],
    )(p, h, W1, b1, W2, b2, fW1, fb1, fW2, fb2, gW1, gb1, gW2r, gb2)


def kernel(edge_index, r, z, node_embedding, edge_table, filt_W, filt_b, src_W,
           out_W1, out_b1, out_W2, out_b2, fc_W1, fc_b1, fc_W2, fc_b2,
           fc2_W1, fc2_b1, fc2_W2, fc2_b2):
    _sc_gather_z, _sc_gather_rows, _sc_scatter_add = _sc_kernels()
    src3 = edge_index[0].reshape(NW, NCH, KE)
    dst3 = edge_index[1].reshape(NW, NCH, KE)
    r3 = r.reshape(NBE, 1, BE)
    zeros_pad = jnp.zeros((N_PAD, H), jnp.float32)

    zs3, zd3 = _sc_gather_z(src3, dst3, z)
    padlen = E_PADROWS * 128 - E
    rpad = jnp.concatenate([r, jnp.zeros((padlen,), jnp.float32)])
    zpad = jnp.zeros((padlen,), jnp.int32)
    zspad = jnp.concatenate([zs3.reshape(-1), zpad])
    zdpad = jnp.concatenate([zd3.reshape(-1), 1 - zpad])
    pk3 = _cut_call(
        rpad.reshape(E_PADROWS, 128),
        zspad.reshape(E_PADROWS, 128),
        zdpad.reshape(E_PADROWS, 128),
    ).reshape(-1)[:E].reshape(NBE, 1, BE)

    def edge_table_args(l):
        return (filt_W[l], filt_b[l].reshape(1, H), edge_table)

    m = _edge_call(r3, pk3, src_W[0], *edge_table_args(0), layer0=True)
    p = _sc_scatter_add(m, dst3, zeros_pad)
    h, hs = _node_call(p, zeros_pad, out_W1[0], out_b1[0].reshape(1, H),
                       out_W2[0], out_b2[0].reshape(1, H), src_W[1], layer0=True)

    g = _sc_gather_rows(hs, src3)
    m = _edge_call(r3, pk3, g, *edge_table_args(1), layer0=False)
    p = _sc_scatter_add(m, dst3, zeros_pad)
    h, hs = _node_call(p, h, out_W1[1], out_b1[1].reshape(1, H),
                       out_W2[1], out_b2[1].reshape(1, H), src_W[2], layer0=False)

    g = _sc_gather_rows(hs, src3)
    m = _edge_call(r3, pk3, g, *edge_table_args(2), layer0=False)
    p = _sc_scatter_add(m, dst3, zeros_pad)

    out = _final_call(
        p, h, out_W1[2], out_b1[2].reshape(1, H), out_W2[2], out_b2[2].reshape(1, H),
        fc_W1, fc_b1.reshape(1, H), fc_W2, fc_b2.reshape(1, H),
        fc2_W1, fc2_b1.reshape(1, H), fc2_W2.reshape(1, H), fc2_b2.reshape(1, 1),
    )
    return out.reshape(1)


# speedup vs baseline: 8.2249x; 1.1499x over previous
import functools
import math

import jax
import jax.numpy as jnp
from jax import lax
from jax.experimental import pallas as pl
from jax.experimental.pallas import tpu as pltpu
from jax.experimental.pallas import tpu_sc as plsc

N = 10000
E = 320000
H = 128
R = 64

N_PAD = 10240

NC = 2
NS = 16
NW = NC * NS
EPW = E // NW
KE = 80
NCH = EPW // KE

EA = 128000
EB = E - EA
NCH_A = EA // NW // KE
NCH_B = EB // NW // KE

BE = 2000
NBE = E // BE
BN = 2048
NBN = N_PAD // BN

_LOG2 = math.log(2.0)
_SUB_ROWS = N_PAD // NS


_LOG2E = 1.4426950408889634


def _ssp(x):
    t = jnp.log2(1.0 + jnp.exp2(-jnp.abs(x) * _LOG2E))
    return jnp.maximum(x, 0.0) + (t - 1.0) * _LOG2


def _wid():
    return lax.axis_index("s") * NC + lax.axis_index("c")


@functools.cache
def _sc_kernels():
    mesh = plsc.VectorSubcoreMesh(
        core_axis_name="c", subcore_axis_name="s", num_cores=NC, num_subcores=NS
    )

    @functools.partial(
        pl.kernel,
        out_type=(
            jax.ShapeDtypeStruct((NW, NCH, KE), jnp.int32),
            jax.ShapeDtypeStruct((NW, NCH, KE), jnp.int32),
        ),
        mesh=mesh,
        scratch_types=[
            pltpu.VMEM((NCH, KE), jnp.int32),
            pltpu.VMEM((NCH, KE), jnp.int32),
            pltpu.VMEM((NCH, KE), jnp.int32),
            pltpu.VMEM((NCH, KE), jnp.int32),
            pltpu.SemaphoreType.DMA,
            pltpu.SemaphoreType.DMA,
        ],
    )
    def _sc_gather_z(src3, dst3, z_hbm, zs3, zd3, sidx, didx, sval, dval, ssem, dsem):
        w = _wid()
        pltpu.sync_copy(src3.at[w], sidx)
        pltpu.sync_copy(dst3.at[w], didx)
        PIPE = 8

        def fire(j, idx, val, sem):
            pltpu.make_async_copy(z_hbm.at[idx.at[j]], val.at[j], sem).start()

        def drain(j, idx, val, sem):
            pltpu.make_async_copy(z_hbm.at[idx.at[j]], val.at[j], sem).wait()

        def body(j, _):
            fire(j, sidx, sval, ssem)
            fire(j, didx, dval, dsem)

            @pl.when(j >= PIPE)
            def _():
                drain(j - PIPE, sidx, sval, ssem)
                drain(j - PIPE, didx, dval, dsem)

            return 0

        lax.fori_loop(0, NCH, body, 0)

        def tail(j, _):
            drain(j, sidx, sval, ssem)
            drain(j, didx, dval, dsem)
            return 0

        lax.fori_loop(NCH - PIPE, NCH, tail, 0)
        pltpu.sync_copy(sval, zs3.at[w])
        pltpu.sync_copy(dval, zd3.at[w])

    NB_G = 5

    def _mk_gather(nch, ep):
      epw = ep // NW
      no_g = nch // NB_G

      @functools.partial(
        pl.kernel,
        out_type=jax.ShapeDtypeStruct((ep, H), jnp.float32),
        mesh=mesh,
        scratch_types=[
            pltpu.VMEM((nch, KE), jnp.int32),
            pltpu.VMEM((KE, H), jnp.float32),
            pltpu.VMEM((KE, H), jnp.float32),
            pltpu.VMEM((KE, H), jnp.float32),
            pltpu.VMEM((KE, H), jnp.float32),
            pltpu.VMEM((KE, H), jnp.float32),
            pltpu.SemaphoreType.DMA,
            pltpu.SemaphoreType.DMA,
            pltpu.SemaphoreType.DMA,
            pltpu.SemaphoreType.DMA,
            pltpu.SemaphoreType.DMA,
            pltpu.SemaphoreType.DMA,
            pltpu.SemaphoreType.DMA,
            pltpu.SemaphoreType.DMA,
            pltpu.SemaphoreType.DMA,
            pltpu.SemaphoreType.DMA,
        ],
    )
      def _sc_gather_rows(hs_hbm, src3, g_hbm, idx_v, b0, b1, b2, b3, b4,
                          g0, g1, g2, g3, g4, w0, w1, w2, w3, w4):
        bufs = (b0, b1, b2, b3, b4)
        gsems = (g0, g1, g2, g3, g4)
        wsems = (w0, w1, w2, w3, w4)
        w = _wid()
        base = w * epw
        pltpu.sync_copy(src3.at[w], idx_v)

        def outer(o, _):
            @pl.when(o > 0)
            def _():
                for b in range(NB_G):
                    pltpu.make_async_copy(
                        bufs[b],
                        g_hbm.at[pl.ds(base + ((o - 1) * NB_G + b) * KE, KE)],
                        wsems[b],
                    ).wait()

            for b in range(NB_G):
                pltpu.make_async_copy(
                    hs_hbm.at[idx_v.at[o * NB_G + b]], bufs[b], gsems[b]
                ).start()
            for b in range(NB_G):
                pltpu.make_async_copy(
                    hs_hbm.at[idx_v.at[o * NB_G + b]], bufs[b], gsems[b]
                ).wait()
                pltpu.make_async_copy(
                    bufs[b],
                    g_hbm.at[pl.ds(base + (o * NB_G + b) * KE, KE)],
                    wsems[b],
                ).start()

            return 0

        lax.fori_loop(0, no_g, outer, 0)
        for b in range(NB_G):
            pltpu.make_async_copy(
                bufs[b],
                g_hbm.at[pl.ds(base + ((no_g - 1) * NB_G + b) * KE, KE)],
                wsems[b],
            ).wait()

      return _sc_gather_rows

    def _mk_scatter(nch, ep):
      epw = ep // NW

      @functools.partial(
        pl.kernel,
        out_type=jax.ShapeDtypeStruct((NC, N_PAD, H), jnp.float32),
        mesh=mesh,
        scratch_types=[
            pltpu.VMEM((nch, KE), jnp.int32),
            pltpu.VMEM((KE, H), jnp.float32),
            pltpu.VMEM((KE, H), jnp.float32),
            pltpu.VMEM_SHARED((N_PAD, H), jnp.float32),
            pltpu.SemaphoreType.DMA,
            pltpu.SemaphoreType.DMA,
        ],
      )
      def _sc_scatter_add(m_hbm, dst3, zero_hbm, out_hbm, idx_v, b0, b1,
                          acc, l0, l1):
        bufs = (b0, b1)
        lsems = (l0, l1)
        c = lax.axis_index("c")
        s = lax.axis_index("s")
        w = s * NC + c
        pltpu.sync_copy(
            zero_hbm.at[pl.ds(s * _SUB_ROWS, _SUB_ROWS)],
            acc.at[pl.ds(s * _SUB_ROWS, _SUB_ROWS)],
        )
        pltpu.sync_copy(dst3.at[w], idx_v)
        plsc.subcore_barrier()
        base = w * epw

        def load(j, slot):
            pltpu.make_async_copy(
                m_hbm.at[pl.ds(base + j * KE, KE)], bufs[slot], lsems[slot]
            ).start()

        def wait_load(j, slot):
            pltpu.make_async_copy(
                m_hbm.at[pl.ds(base + j * KE, KE)], bufs[slot], lsems[slot]
            ).wait()

        load(0, 0)

        def body(j, _):
            @pl.when(j % 2 == 0)
            def _():
                @pl.when(j + 1 < nch)
                def _():
                    load(j + 1, 1)

                wait_load(j, 0)
                pltpu.sync_copy(bufs[0], acc.at[idx_v.at[j]], add=True)

            @pl.when(j % 2 == 1)
            def _():
                @pl.when(j + 1 < nch)
                def _():
                    load(j + 1, 0)

                wait_load(j, 1)
                pltpu.sync_copy(bufs[1], acc.at[idx_v.at[j]], add=True)

            return 0

        lax.fori_loop(0, nch, body, 0)
        plsc.subcore_barrier()
        pltpu.sync_copy(
            acc.at[pl.ds(s * _SUB_ROWS, _SUB_ROWS)],
            out_hbm.at[c, pl.ds(s * _SUB_ROWS, _SUB_ROWS)],
        )

      return _sc_scatter_add

    return (
        _sc_gather_z,
        _mk_gather(NCH_A, EA),
        _mk_gather(NCH_B, EB),
        _mk_scatter(NCH_A, EA),
        _mk_scatter(NCH_B, EB),
    )


def _cut_body(r_ref, zs_ref, zd_ref, pk_ref):
    r = r_ref[...]
    cut = jnp.where(r < 1.0, 0.5 * (jnp.cos(math.pi * r) + 1.0), 0.0)
    pk_ref[...] = jnp.where(zs_ref[...] == zd_ref[...], -cut, cut)


E_PADROWS = (E + 511) // 512 * 4


def _cut_call(r2d, zs2d, zd2d):
    return pl.pallas_call(
        _cut_body,
        out_shape=jax.ShapeDtypeStruct((E_PADROWS, 128), jnp.float32),
    )(r2d, zs2d, zd2d)


def _edge_body(r_ref, pk_ref, g_ref, fW_ref, fb_ref, et_ref, m_ref, *, layer0):
    r = r_ref[0, 0, :].reshape(1, BE)
    pc = pk_ref[0, 0, :].reshape(BE, 1)
    cutc = jnp.abs(pc)
    delta = 1.0 / (R - 1)
    centers = lax.broadcasted_iota(jnp.int32, (R, BE), 0).astype(jnp.float32) * delta
    t = (r - centers) * (1.0 / delta)
    bfT = jnp.exp2(t * t * (-0.5 * _LOG2E)) * jnp.abs(pk_ref[0, 0, :].reshape(1, BE))
    filt = _ssp(
        lax.dot_general(bfT, fW_ref[...], (((0,), (0,)), ((), ())),
                        preferred_element_type=jnp.float32)
        + fb_ref[...]
    )
    eh = jnp.where(pc < 0.0, et_ref[1, :][None, :], et_ref[0, :][None, :])
    if layer0:
        g = jnp.sum(g_ref[...], axis=0, keepdims=True)
    else:
        g = g_ref[...]
    m_ref[...] = (g + eh) * filt * cutc


def _edge_call(r3, pk3, g_or_w, fW, fb, et, *, layer0):
    ep = r3.shape[0] * BE
    vec_spec = pl.BlockSpec((1, 1, BE), lambda i: (i, 0, 0))
    g_spec = (
        pl.BlockSpec((H, H), lambda i: (0, 0))
        if layer0
        else pl.BlockSpec((BE, H), lambda i: (i, 0))
    )
    return pl.pallas_call(
        functools.partial(_edge_body, layer0=layer0),
        grid=(ep // BE,),
        in_specs=[
            vec_spec,
            vec_spec,
            g_spec,
            pl.BlockSpec((R, H), lambda i: (0, 0)),
            pl.BlockSpec((1, H), lambda i: (0, 0)),
            pl.BlockSpec((2, H), lambda i: (0, 0)),
        ],
        out_specs=pl.BlockSpec((BE, H), lambda i: (i, 0)),
        out_shape=jax.ShapeDtypeStruct((ep, H), jnp.float32),
    )(r3, pk3, g_or_w, fW, fb, et)


def _node_body(p_ref, q_ref, h_ref, W1_ref, b1_ref, W2_ref, b2_ref, sW_ref, hnew_ref, hs_ref, *, layer0):
    agg = (p_ref[0] + p_ref[1]) + (q_ref[0] + q_ref[1])
    u = _ssp(jnp.dot(agg, W1_ref[...], preferred_element_type=jnp.float32) + b1_ref[...])
    upd = jnp.dot(u, W2_ref[...], preferred_element_type=jnp.float32) + b2_ref[...]
    if layer0:
        hnew = 1.0 + upd
    else:
        hnew = h_ref[...] + upd
    hnew_ref[...] = hnew
    hs_ref[...] = jnp.dot(hnew, sW_ref[...], preferred_element_type=jnp.float32)


def _node_call(p, q, h, W1, b1, W2, b2, sW_next, *, layer0):
    full = lambda shape: pl.BlockSpec(shape, lambda i: tuple(0 for _ in shape))
    in_specs = [
        pl.BlockSpec((NC, BN, H), lambda i: (0, i, 0)),
        pl.BlockSpec((NC, BN, H), lambda i: (0, i, 0)),
        pl.BlockSpec((BN, H), lambda i: (i, 0)),
        full((H, H)),
        full((1, H)),
        full((H, H)),
        full((1, H)),
        full((H, H)),
    ]
    args = [p, q, h, W1, b1, W2, b2, sW_next]
    return pl.pallas_call(
        functools.partial(_node_body, layer0=layer0),
        grid=(NBN,),
        in_specs=in_specs,
        out_specs=(
            pl.BlockSpec((BN, H), lambda i: (i, 0)),
            pl.BlockSpec((BN, H), lambda i: (i, 0)),
        ),
        out_shape=(
            jax.ShapeDtypeStruct((N_PAD, H), jnp.float32),
            jax.ShapeDtypeStruct((N_PAD, H), jnp.float32),
        ),
    )(*args)


def _final_body(p_ref, q_ref, h_ref, W1_ref, b1_ref, W2_ref, b2_ref,
                fW1_ref, fb1_ref, fW2_ref, fb2_ref,
                gW1_ref, gb1_ref, gW2r_ref, gb2_ref, out_ref, acc_ref):
    i = pl.program_id(0)
    agg = (p_ref[0] + p_ref[1]) + (q_ref[0] + q_ref[1])
    u = _ssp(jnp.dot(agg, W1_ref[...], preferred_element_type=jnp.float32) + b1_ref[...])
    upd = jnp.dot(u, W2_ref[...], preferred_element_type=jnp.float32) + b2_ref[...]
    hnew = h_ref[...] + upd
    t = _ssp(jnp.dot(hnew, fW1_ref[...], preferred_element_type=jnp.float32) + fb1_ref[...])
    row = lax.broadcasted_iota(jnp.int32, (BN, 1), 0) + i * BN
    t = jnp.where(row < N, t, 0.0)
    part = jnp.sum(t, axis=0, keepdims=True)

    @pl.when(i == 0)
    def _():
        acc_ref[...] = part

    @pl.when(i > 0)
    def _():
        acc_ref[...] = acc_ref[...] + part

    @pl.when(i == NBN - 1)
    def _():
        pooled = (
            jnp.dot(acc_ref[...] * (1.0 / N), fW2_ref[...],
                    preferred_element_type=jnp.float32)
            + fb2_ref[...]
        )
        o = _ssp(
            jnp.dot(pooled, gW1_ref[...], preferred_element_type=jnp.float32)
            + gb1_ref[...]
        )
        out_ref[...] = (
            jnp.sum(o * gW2r_ref[...], axis=1, keepdims=True) + gb2_ref[...]
        )


def _final_call(p, q, h, W1, b1, W2, b2, fW1, fb1, fW2, fb2, gW1, gb1, gW2r, gb2):
    full = lambda shape: pl.BlockSpec(shape, lambda i: tuple(0 for _ in shape))
    return pl.pallas_call(
        _final_body,
        grid=(NBN,),
        in_specs=[
            pl.BlockSpec((NC, BN, H), lambda i: (0, i, 0)),
            pl.BlockSpec((NC, BN, H), lambda i: (0, i, 0)),
            pl.BlockSpec((BN, H), lambda i: (i, 0)),
            full((H, H)), full((1, H)), full((H, H)), full((1, H)),
            full((H, H)), full((1, H)), full((H, H)), full((1, H)),
            full((H, H)), full((1, H)), full((1, H)), full((1, 1)),
        ],
        out_specs=pl.BlockSpec((1, 1), lambda i: (0, 0)),
        out_shape=jax.ShapeDtypeStruct((1, 1), jnp.float32),
        scratch_shapes=[pltpu.VMEM((1, H), jnp.float32)],
    )(p, q, h, W1, b1, W2, b2, fW1, fb1, fW2, fb2, gW1, gb1, gW2r, gb2)


def kernel(edge_index, r, z, node_embedding, edge_table, filt_W, filt_b, src_W,
           out_W1, out_b1, out_W2, out_b2, fc_W1, fc_b1, fc_W2, fc_b2,
           fc2_W1, fc2_b1, fc2_W2, fc2_b2):
    (_sc_gather_z, _sc_gather_A, _sc_gather_B,
     _sc_scatter_A, _sc_scatter_B) = _sc_kernels()
    src = edge_index[0]
    dst = edge_index[1]
    src3 = src.reshape(NW, NCH, KE)
    dst3 = dst.reshape(NW, NCH, KE)
    srcA3 = src[:EA].reshape(NW, NCH_A, KE)
    srcB3 = src[EA:].reshape(NW, NCH_B, KE)
    dstA3 = dst[:EA].reshape(NW, NCH_A, KE)
    dstB3 = dst[EA:].reshape(NW, NCH_B, KE)
    rA3 = r[:EA].reshape(EA // BE, 1, BE)
    rB3 = r[EA:].reshape(EB // BE, 1, BE)
    zeros_pad = jnp.zeros((N_PAD, H), jnp.float32)

    zs3, zd3 = _sc_gather_z(src3, dst3, z)
    padlen = E_PADROWS * 128 - E
    rpad = jnp.concatenate([r, jnp.zeros((padlen,), jnp.float32)])
    zpad = jnp.zeros((padlen,), jnp.int32)
    zspad = jnp.concatenate([zs3.reshape(-1), zpad])
    zdpad = jnp.concatenate([zd3.reshape(-1), 1 - zpad])
    pk = _cut_call(
        rpad.reshape(E_PADROWS, 128),
        zspad.reshape(E_PADROWS, 128),
        zdpad.reshape(E_PADROWS, 128),
    ).reshape(-1)
    pkA3 = pk[:EA].reshape(EA // BE, 1, BE)
    pkB3 = pk[EA:E].reshape(EB // BE, 1, BE)

    def edge_table_args(l):
        return (filt_W[l], filt_b[l].reshape(1, H), edge_table)

    mA = _edge_call(rA3, pkA3, src_W[0], *edge_table_args(0), layer0=True)
    pA = _sc_scatter_A(mA, dstA3, zeros_pad)
    mB = _edge_call(rB3, pkB3, src_W[0], *edge_table_args(0), layer0=True)
    pB = _sc_scatter_B(mB, dstB3, zeros_pad)
    h, hs = _node_call(pA, pB, zeros_pad, out_W1[0], out_b1[0].reshape(1, H),
                       out_W2[0], out_b2[0].reshape(1, H), src_W[1], layer0=True)

    for l in (1, 2):
        gA = _sc_gather_A(hs, srcA3)
        mA = _edge_call(rA3, pkA3, gA, *edge_table_args(l), layer0=False)
        gB = _sc_gather_B(hs, srcB3)
        mB = _edge_call(rB3, pkB3, gB, *edge_table_args(l), layer0=False)
        pA = _sc_scatter_A(mA, dstA3, zeros_pad)
        pB = _sc_scatter_B(mB, dstB3, zeros_pad)
        if l == 1:
            h, hs = _node_call(pA, pB, h, out_W1[1], out_b1[1].reshape(1, H),
                               out_W2[1], out_b2[1].reshape(1, H), src_W[2],
                               layer0=False)

    out = _final_call(
        pA, pB, h, out_W1[2], out_b1[2].reshape(1, H), out_W2[2], out_b2[2].reshape(1, H),
        fc_W1, fc_b1.reshape(1, H), fc_W2, fc_b2.reshape(1, H),
        fc2_W1, fc2_b1.reshape(1, H), fc2_W2.reshape(1, H), fc2_b2.reshape(1, 1),
    )
    return out.reshape(1)


# speedup vs baseline: 8.2876x; 1.0076x over previous
import functools
import math

import jax
import jax.numpy as jnp
from jax import lax
from jax.experimental import pallas as pl
from jax.experimental.pallas import tpu as pltpu
from jax.experimental.pallas import tpu_sc as plsc

N = 10000
E = 320000
H = 128
R = 64

N_PAD = 10240

NC = 2
NS = 16
NW = NC * NS
EPW = E // NW
KE = 80
NCH = EPW // KE

PARTS = (64000, 128000, 128000)
P_OFF = (0, 64000, 192000)
NCH_P = tuple(p // (32 * 80) for p in PARTS)
EZA = 128000
NCH_ZA = EZA // (32 * 80)
NCH_ZB = (E - EZA) // (32 * 80)

BE = 2000
NBE = E // BE
BN = 2048
NBN = N_PAD // BN

_LOG2 = math.log(2.0)
_SUB_ROWS = N_PAD // NS


_LOG2E = 1.4426950408889634


def _ssp(x):
    t = jnp.log2(1.0 + jnp.exp2(-jnp.abs(x) * _LOG2E))
    return jnp.maximum(x, 0.0) + (t - 1.0) * _LOG2


def _wid():
    return lax.axis_index("s") * NC + lax.axis_index("c")


@functools.cache
def _sc_kernels():
    mesh = plsc.VectorSubcoreMesh(
        core_axis_name="c", subcore_axis_name="s", num_cores=NC, num_subcores=NS
    )

    def _mk_gather_z(nch):
      @functools.partial(
        pl.kernel,
        out_type=(
            jax.ShapeDtypeStruct((NW, nch, KE), jnp.int32),
            jax.ShapeDtypeStruct((NW, nch, KE), jnp.int32),
        ),
        mesh=mesh,
        scratch_types=[
            pltpu.VMEM((nch, KE), jnp.int32),
            pltpu.VMEM((nch, KE), jnp.int32),
            pltpu.VMEM((nch, KE), jnp.int32),
            pltpu.VMEM((nch, KE), jnp.int32),
            pltpu.SemaphoreType.DMA,
            pltpu.SemaphoreType.DMA,
        ],
      )
      def _sc_gather_z(src3, dst3, z_hbm, zs3, zd3, sidx, didx, sval, dval, ssem, dsem):
        w = _wid()
        pltpu.sync_copy(src3.at[w], sidx)
        pltpu.sync_copy(dst3.at[w], didx)
        PIPE = 8

        def fire(j, idx, val, sem):
            pltpu.make_async_copy(z_hbm.at[idx.at[j]], val.at[j], sem).start()

        def drain(j, idx, val, sem):
            pltpu.make_async_copy(z_hbm.at[idx.at[j]], val.at[j], sem).wait()

        def body(j, _):
            fire(j, sidx, sval, ssem)
            fire(j, didx, dval, dsem)

            @pl.when(j >= PIPE)
            def _():
                drain(j - PIPE, sidx, sval, ssem)
                drain(j - PIPE, didx, dval, dsem)

            return 0

        lax.fori_loop(0, nch, body, 0)

        def tail(j, _):
            drain(j, sidx, sval, ssem)
            drain(j, didx, dval, dsem)
            return 0

        lax.fori_loop(nch - PIPE, nch, tail, 0)
        pltpu.sync_copy(sval, zs3.at[w])
        pltpu.sync_copy(dval, zd3.at[w])

      return _sc_gather_z

    NB_G = 5

    def _mk_gather(nch, ep):
      epw = ep // NW
      no_g = nch // NB_G

      @functools.partial(
        pl.kernel,
        out_type=jax.ShapeDtypeStruct((ep, H), jnp.float32),
        mesh=mesh,
        scratch_types=[
            pltpu.VMEM((nch, KE), jnp.int32),
            pltpu.VMEM((KE, H), jnp.float32),
            pltpu.VMEM((KE, H), jnp.float32),
            pltpu.VMEM((KE, H), jnp.float32),
            pltpu.VMEM((KE, H), jnp.float32),
            pltpu.VMEM((KE, H), jnp.float32),
            pltpu.SemaphoreType.DMA,
            pltpu.SemaphoreType.DMA,
            pltpu.SemaphoreType.DMA,
            pltpu.SemaphoreType.DMA,
            pltpu.SemaphoreType.DMA,
            pltpu.SemaphoreType.DMA,
            pltpu.SemaphoreType.DMA,
            pltpu.SemaphoreType.DMA,
            pltpu.SemaphoreType.DMA,
            pltpu.SemaphoreType.DMA,
        ],
    )
      def _sc_gather_rows(hs_hbm, src3, g_hbm, idx_v, b0, b1, b2, b3, b4,
                          g0, g1, g2, g3, g4, w0, w1, w2, w3, w4):
        bufs = (b0, b1, b2, b3, b4)
        gsems = (g0, g1, g2, g3, g4)
        wsems = (w0, w1, w2, w3, w4)
        w = _wid()
        base = w * epw
        pltpu.sync_copy(src3.at[w], idx_v)

        def outer(o, _):
            @pl.when(o > 0)
            def _():
                for b in range(NB_G):
                    pltpu.make_async_copy(
                        bufs[b],
                        g_hbm.at[pl.ds(base + ((o - 1) * NB_G + b) * KE, KE)],
                        wsems[b],
                    ).wait()

            for b in range(NB_G):
                pltpu.make_async_copy(
                    hs_hbm.at[idx_v.at[o * NB_G + b]], bufs[b], gsems[b]
                ).start()
            for b in range(NB_G):
                pltpu.make_async_copy(
                    hs_hbm.at[idx_v.at[o * NB_G + b]], bufs[b], gsems[b]
                ).wait()
                pltpu.make_async_copy(
                    bufs[b],
                    g_hbm.at[pl.ds(base + (o * NB_G + b) * KE, KE)],
                    wsems[b],
                ).start()

            return 0

        lax.fori_loop(0, no_g, outer, 0)
        for b in range(NB_G):
            pltpu.make_async_copy(
                bufs[b],
                g_hbm.at[pl.ds(base + ((no_g - 1) * NB_G + b) * KE, KE)],
                wsems[b],
            ).wait()

      return _sc_gather_rows

    def _mk_scatter(nch, ep):
      epw = ep // NW

      @functools.partial(
        pl.kernel,
        out_type=jax.ShapeDtypeStruct((NC, N_PAD, H), jnp.float32),
        mesh=mesh,
        scratch_types=[
            pltpu.VMEM((nch, KE), jnp.int32),
            pltpu.VMEM((KE, H), jnp.float32),
            pltpu.VMEM((KE, H), jnp.float32),
            pltpu.VMEM_SHARED((N_PAD, H), jnp.float32),
            pltpu.SemaphoreType.DMA,
            pltpu.SemaphoreType.DMA,
        ],
      )
      def _sc_scatter_add(m_hbm, dst3, zero_hbm, out_hbm, idx_v, b0, b1,
                          acc, l0, l1):
        bufs = (b0, b1)
        lsems = (l0, l1)
        c = lax.axis_index("c")
        s = lax.axis_index("s")
        w = s * NC + c
        pltpu.sync_copy(
            zero_hbm.at[pl.ds(s * _SUB_ROWS, _SUB_ROWS)],
            acc.at[pl.ds(s * _SUB_ROWS, _SUB_ROWS)],
        )
        pltpu.sync_copy(dst3.at[w], idx_v)
        plsc.subcore_barrier()
        base = w * epw

        def load(j, slot):
            pltpu.make_async_copy(
                m_hbm.at[pl.ds(base + j * KE, KE)], bufs[slot], lsems[slot]
            ).start()

        def wait_load(j, slot):
            pltpu.make_async_copy(
                m_hbm.at[pl.ds(base + j * KE, KE)], bufs[slot], lsems[slot]
            ).wait()

        load(0, 0)

        def body(j, _):
            @pl.when(j % 2 == 0)
            def _():
                @pl.when(j + 1 < nch)
                def _():
                    load(j + 1, 1)

                wait_load(j, 0)
                pltpu.sync_copy(bufs[0], acc.at[idx_v.at[j]], add=True)

            @pl.when(j % 2 == 1)
            def _():
                @pl.when(j + 1 < nch)
                def _():
                    load(j + 1, 0)

                wait_load(j, 1)
                pltpu.sync_copy(bufs[1], acc.at[idx_v.at[j]], add=True)

            return 0

        lax.fori_loop(0, nch, body, 0)
        plsc.subcore_barrier()
        pltpu.sync_copy(
            acc.at[pl.ds(s * _SUB_ROWS, _SUB_ROWS)],
            out_hbm.at[c, pl.ds(s * _SUB_ROWS, _SUB_ROWS)],
        )

      return _sc_scatter_add

    return (
        (_mk_gather_z(NCH_ZA), _mk_gather_z(NCH_ZB)),
        tuple(_mk_gather(NCH_P[i], PARTS[i]) for i in range(3)),
        tuple(_mk_scatter(NCH_P[i], PARTS[i]) for i in range(3)),
    )


def _cut_body(r_ref, zs_ref, zd_ref, pk_ref):
    r = r_ref[...]
    cut = jnp.where(r < 1.0, 0.5 * (jnp.cos(math.pi * r) + 1.0), 0.0)
    pk_ref[...] = jnp.where(zs_ref[...] == zd_ref[...], -cut, cut)


E_PADROWS = (E + 511) // 512 * 4


def _cut_call(r2d, zs2d, zd2d):
    return pl.pallas_call(
        _cut_body,
        out_shape=jax.ShapeDtypeStruct((E_PADROWS, 128), jnp.float32),
    )(r2d, zs2d, zd2d)


def _edge_body(r_ref, pk_ref, g_ref, fW_ref, fb_ref, et_ref, m_ref, *, layer0):
    r = r_ref[0, 0, :].reshape(1, BE)
    pc = pk_ref[0, 0, :].reshape(BE, 1)
    cutc = jnp.abs(pc)
    delta = 1.0 / (R - 1)
    centers = lax.broadcasted_iota(jnp.int32, (R, BE), 0).astype(jnp.float32) * delta
    t = (r - centers) * (1.0 / delta)
    bfT = jnp.exp2(t * t * (-0.5 * _LOG2E)) * jnp.abs(pk_ref[0, 0, :].reshape(1, BE))
    filt = _ssp(
        lax.dot_general(bfT, fW_ref[...], (((0,), (0,)), ((), ())),
                        preferred_element_type=jnp.float32)
        + fb_ref[...]
    )
    eh = jnp.where(pc < 0.0, et_ref[1, :][None, :], et_ref[0, :][None, :])
    if layer0:
        g = jnp.sum(g_ref[...], axis=0, keepdims=True)
    else:
        g = g_ref[...]
    m_ref[...] = (g + eh) * filt * cutc


def _edge_call(r3, pk3, g_or_w, fW, fb, et, *, layer0):
    ep = r3.shape[0] * BE
    vec_spec = pl.BlockSpec((1, 1, BE), lambda i: (i, 0, 0))
    g_spec = (
        pl.BlockSpec((H, H), lambda i: (0, 0))
        if layer0
        else pl.BlockSpec((BE, H), lambda i: (i, 0))
    )
    return pl.pallas_call(
        functools.partial(_edge_body, layer0=layer0),
        grid=(ep // BE,),
        in_specs=[
            vec_spec,
            vec_spec,
            g_spec,
            pl.BlockSpec((R, H), lambda i: (0, 0)),
            pl.BlockSpec((1, H), lambda i: (0, 0)),
            pl.BlockSpec((2, H), lambda i: (0, 0)),
        ],
        out_specs=pl.BlockSpec((BE, H), lambda i: (i, 0)),
        out_shape=jax.ShapeDtypeStruct((ep, H), jnp.float32),
    )(r3, pk3, g_or_w, fW, fb, et)


def _node_body(p_ref, q_ref, t_ref, h_ref, W1_ref, b1_ref, W2_ref, b2_ref, sW_ref, hnew_ref, hs_ref, *, layer0):
    agg = (p_ref[0] + p_ref[1]) + (q_ref[0] + q_ref[1]) + (t_ref[0] + t_ref[1])
    u = _ssp(jnp.dot(agg, W1_ref[...], preferred_element_type=jnp.float32) + b1_ref[...])
    upd = jnp.dot(u, W2_ref[...], preferred_element_type=jnp.float32) + b2_ref[...]
    if layer0:
        hnew = 1.0 + upd
    else:
        hnew = h_ref[...] + upd
    hnew_ref[...] = hnew
    hs_ref[...] = jnp.dot(hnew, sW_ref[...], preferred_element_type=jnp.float32)


def _node_call(p, q, t, h, W1, b1, W2, b2, sW_next, *, layer0):
    full = lambda shape: pl.BlockSpec(shape, lambda i: tuple(0 for _ in shape))
    in_specs = [
        pl.BlockSpec((NC, BN, H), lambda i: (0, i, 0)),
        pl.BlockSpec((NC, BN, H), lambda i: (0, i, 0)),
        pl.BlockSpec((NC, BN, H), lambda i: (0, i, 0)),
        pl.BlockSpec((BN, H), lambda i: (i, 0)),
        full((H, H)),
        full((1, H)),
        full((H, H)),
        full((1, H)),
        full((H, H)),
    ]
    args = [p, q, t, h, W1, b1, W2, b2, sW_next]
    return pl.pallas_call(
        functools.partial(_node_body, layer0=layer0),
        grid=(NBN,),
        in_specs=in_specs,
        out_specs=(
            pl.BlockSpec((BN, H), lambda i: (i, 0)),
            pl.BlockSpec((BN, H), lambda i: (i, 0)),
        ),
        out_shape=(
            jax.ShapeDtypeStruct((N_PAD, H), jnp.float32),
            jax.ShapeDtypeStruct((N_PAD, H), jnp.float32),
        ),
    )(*args)


def _final_body(p_ref, q_ref, t_ref, h_ref, W1_ref, b1_ref, W2_ref, b2_ref,
                fW1_ref, fb1_ref, fW2_ref, fb2_ref,
                gW1_ref, gb1_ref, gW2r_ref, gb2_ref, out_ref, acc_ref):
    i = pl.program_id(0)
    agg = (p_ref[0] + p_ref[1]) + (q_ref[0] + q_ref[1]) + (t_ref[0] + t_ref[1])
    u = _ssp(jnp.dot(agg, W1_ref[...], preferred_element_type=jnp.float32) + b1_ref[...])
    upd = jnp.dot(u, W2_ref[...], preferred_element_type=jnp.float32) + b2_ref[...]
    hnew = h_ref[...] + upd
    t = _ssp(jnp.dot(hnew, fW1_ref[...], preferred_element_type=jnp.float32) + fb1_ref[...])
    row = lax.broadcasted_iota(jnp.int32, (BN, 1), 0) + i * BN
    t = jnp.where(row < N, t, 0.0)
    part = jnp.sum(t, axis=0, keepdims=True)

    @pl.when(i == 0)
    def _():
        acc_ref[...] = part

    @pl.when(i > 0)
    def _():
        acc_ref[...] = acc_ref[...] + part

    @pl.when(i == NBN - 1)
    def _():
        pooled = (
            jnp.dot(acc_ref[...] * (1.0 / N), fW2_ref[...],
                    preferred_element_type=jnp.float32)
            + fb2_ref[...]
        )
        o = _ssp(
            jnp.dot(pooled, gW1_ref[...], preferred_element_type=jnp.float32)
            + gb1_ref[...]
        )
        out_ref[...] = (
            jnp.sum(o * gW2r_ref[...], axis=1, keepdims=True) + gb2_ref[...]
        )


def _final_call(p, q, t, h, W1, b1, W2, b2, fW1, fb1, fW2, fb2, gW1, gb1, gW2r, gb2):
    full = lambda shape: pl.BlockSpec(shape, lambda i: tuple(0 for _ in shape))
    return pl.pallas_call(
        _final_body,
        grid=(NBN,),
        in_specs=[
            pl.BlockSpec((NC, BN, H), lambda i: (0, i, 0)),
            pl.BlockSpec((NC, BN, H), lambda i: (0, i, 0)),
            pl.BlockSpec((NC, BN, H), lambda i: (0, i, 0)),
            pl.BlockSpec((BN, H), lambda i: (i, 0)),
            full((H, H)), full((1, H)), full((H, H)), full((1, H)),
            full((H, H)), full((1, H)), full((H, H)), full((1, H)),
            full((H, H)), full((1, H)), full((1, H)), full((1, 1)),
        ],
        out_specs=pl.BlockSpec((1, 1), lambda i: (0, 0)),
        out_shape=jax.ShapeDtypeStruct((1, 1), jnp.float32),
        scratch_shapes=[pltpu.VMEM((1, H), jnp.float32)],
    )(p, q, t, h, W1, b1, W2, b2, fW1, fb1, fW2, fb2, gW1, gb1, gW2r, gb2)


def kernel(edge_index, r, z, node_embedding, edge_table, filt_W, filt_b, src_W,
           out_W1, out_b1, out_W2, out_b2, fc_W1, fc_b1, fc_W2, fc_b2,
           fc2_W1, fc2_b1, fc2_W2, fc2_b2):
    (gz_kernels, gather_kernels, scatter_kernels) = _sc_kernels()
    src = edge_index[0]
    dst = edge_index[1]
    zeros_pad = jnp.zeros((N_PAD, H), jnp.float32)

    def part3(a, i, blk):
        lo, n = P_OFF[i], PARTS[i]
        return a[lo : lo + n].reshape(-1, *blk)

    zsA, zdA = gz_kernels[0](src[:EZA].reshape(NW, NCH_ZA, KE),
                             dst[:EZA].reshape(NW, NCH_ZA, KE), z)
    zsB, zdB = gz_kernels[1](src[EZA:].reshape(NW, NCH_ZB, KE),
                             dst[EZA:].reshape(NW, NCH_ZB, KE), z)
    padlen = E_PADROWS * 128 - E
    rpad = jnp.concatenate([r, jnp.zeros((padlen,), jnp.float32)])
    zpad = jnp.zeros((padlen,), jnp.int32)
    zspad = jnp.concatenate([zsA.reshape(-1), zsB.reshape(-1), zpad])
    zdpad = jnp.concatenate([zdA.reshape(-1), zdB.reshape(-1), 1 - zpad])
    pk = _cut_call(
        rpad.reshape(E_PADROWS, 128),
        zspad.reshape(E_PADROWS, 128),
        zdpad.reshape(E_PADROWS, 128),
    ).reshape(-1)[:E]

    r3p = [part3(r, i, (1, BE)) for i in range(3)]
    pk3p = [part3(pk, i, (1, BE)) for i in range(3)]
    src3p = [part3(src, i, (NCH_P[i], KE)).reshape(NW, NCH_P[i], KE) for i in range(3)]
    dst3p = [part3(dst, i, (NCH_P[i], KE)).reshape(NW, NCH_P[i], KE) for i in range(3)]

    def edge_table_args(l):
        return (filt_W[l], filt_b[l].reshape(1, H), edge_table)

    ps = []
    for i in range(3):
        m = _edge_call(r3p[i], pk3p[i], src_W[0], *edge_table_args(0), layer0=True)
        ps.append(scatter_kernels[i](m, dst3p[i], zeros_pad))
    h, hs = _node_call(*ps, zeros_pad, out_W1[0], out_b1[0].reshape(1, H),
                       out_W2[0], out_b2[0].reshape(1, H), src_W[1], layer0=True)

    for l in (1, 2):
        ms = []
        for i in range(3):
            g = gather_kernels[i](hs, src3p[i])
            ms.append(_edge_call(r3p[i], pk3p[i], g, *edge_table_args(l), layer0=False))
        ps = [scatter_kernels[i](ms[i], dst3p[i], zeros_pad) for i in range(3)]
        if l == 1:
            h, hs = _node_call(*ps, h, out_W1[1], out_b1[1].reshape(1, H),
                               out_W2[1], out_b2[1].reshape(1, H), src_W[2],
                               layer0=False)

    out = _final_call(
        *ps, h, out_W1[2], out_b1[2].reshape(1, H), out_W2[2], out_b2[2].reshape(1, H),
        fc_W1, fc_b1.reshape(1, H), fc_W2, fc_b2.reshape(1, H),
        fc2_W1, fc2_b1.reshape(1, H), fc2_W2.reshape(1, H), fc2_b2.reshape(1, 1),
    )
    return out.reshape(1)
